# Initial kernel scaffold; baseline (speedup 1.0000x reference)
import jax
import jax.numpy as jnp
from jax.experimental import pallas as pl


def kernel(hidden_states, gate_w, w1, w2, w3):
    raise NotImplementedError("write your pallas kernel here")


# speedup vs baseline: 1.1057x; 1.1057x over previous
import functools

import jax
import jax.numpy as jnp
from jax.experimental import pallas as pl
from jax.experimental.pallas import tpu as pltpu

BATCH = 2
SEQ = 8192
HIDDEN = 1024
FFN = 3584
NUM_EXPERTS = 8
TOP_K = 2

T = BATCH * SEQ
LANES = 128


_RTM = 2048


def _router_body(x_ref, gw_ref, wd_ref, xbf_ref, idx_ref, wts_ref):
    x = x_ref[...]
    logits = jax.lax.dot_general(
        x, gw_ref[...], (((1,), (1,)), ((), ())),
        preferred_element_type=jnp.float32)
    lane = jax.lax.broadcasted_iota(jnp.int32, logits.shape, 1)
    neg = jnp.float32(-1e30)
    l0 = jnp.where(lane < NUM_EXPERTS, logits, neg)
    m1 = jnp.max(l0, axis=1, keepdims=True)
    i1 = jnp.min(jnp.where(l0 == m1, lane, LANES), axis=1, keepdims=True)
    l1 = jnp.where(lane == i1, neg, l0)
    m2 = jnp.max(l1, axis=1, keepdims=True)
    i2 = jnp.min(jnp.where(l1 == m2, lane, LANES), axis=1, keepdims=True)
    wa = 1.0 / (1.0 + jnp.exp(m2 - m1))
    wb = 1.0 - wa
    wd_ref[...] = jnp.where(lane == i1, wa, 0.0) + jnp.where(lane == i2, wb, 0.0)
    idx_ref[...] = jnp.where(lane == 0, i1, jnp.where(lane == 1, i2, 0))
    wts_ref[...] = jnp.where(lane == 0, wa, jnp.where(lane == 1, wb, 0.0))
    xbf_ref[...] = x.astype(jnp.bfloat16)


def _run_router(x, gate_wp):
    grid = (T // _RTM,)
    return pl.pallas_call(
        _router_body,
        grid=grid,
        in_specs=[
            pl.BlockSpec((_RTM, HIDDEN), lambda m: (m, 0)),
            pl.BlockSpec((LANES, HIDDEN), lambda m: (0, 0)),
        ],
        out_specs=[
            pl.BlockSpec((_RTM, LANES), lambda m: (m, 0)),
            pl.BlockSpec((_RTM, HIDDEN), lambda m: (m, 0)),
            pl.BlockSpec((_RTM, LANES), lambda m: (m, 0)),
            pl.BlockSpec((_RTM, LANES), lambda m: (m, 0)),
        ],
        out_shape=[
            jax.ShapeDtypeStruct((T, LANES), jnp.float32),
            jax.ShapeDtypeStruct((T, HIDDEN), jnp.bfloat16),
            jax.ShapeDtypeStruct((T, LANES), jnp.int32),
            jax.ShapeDtypeStruct((T, LANES), jnp.float32),
        ],
    )(x, gate_wp)


_TM = 2048
_TF = 512


def _dense_body(xbf_ref, w1_ref, w3_ref, w2_ref, wd_ref, out_ref):
    e = pl.program_id(1)
    f = pl.program_id(2)
    xb = xbf_ref[...]
    w1s = w1_ref[0]
    w3s = w3_ref[0]
    w2s = w2_ref[0]
    a = jax.lax.dot_general(xb, w1s, (((1,), (1,)), ((), ())),
                            preferred_element_type=jnp.float32)
    b = jax.lax.dot_general(xb, w3s, (((1,), (1,)), ((), ())),
                            preferred_element_type=jnp.float32)
    h = (a * jax.nn.sigmoid(a) * b).astype(jnp.bfloat16)
    part = jax.lax.dot_general(h, w2s, (((1,), (1,)), ((), ())),
                               preferred_element_type=jnp.float32)
    lane = jax.lax.broadcasted_iota(jnp.int32, (_TM, LANES), 1)
    w_e = jnp.sum(jnp.where(lane == e, wd_ref[...], 0.0), axis=1, keepdims=True)
    contrib = part * w_e

    @pl.when((e == 0) & (f == 0))
    def _():
        out_ref[...] = contrib

    @pl.when((e > 0) | (f > 0))
    def _():
        out_ref[...] = out_ref[...] + contrib


def _run_dense(xbf, w1b, w3b, w2b, wd):
    grid = (T // _TM, NUM_EXPERTS, FFN // _TF)
    return pl.pallas_call(
        _dense_body,
        grid=grid,
        in_specs=[
            pl.BlockSpec((_TM, HIDDEN), lambda m, e, f: (m, 0)),
            pl.BlockSpec((1, _TF, HIDDEN), lambda m, e, f: (e, f, 0)),
            pl.BlockSpec((1, _TF, HIDDEN), lambda m, e, f: (e, f, 0)),
            pl.BlockSpec((1, HIDDEN, _TF), lambda m, e, f: (e, 0, f)),
            pl.BlockSpec((_TM, LANES), lambda m, e, f: (m, 0)),
        ],
        out_specs=pl.BlockSpec((_TM, HIDDEN), lambda m, e, f: (m, 0)),
        out_shape=jax.ShapeDtypeStruct((T, HIDDEN), jnp.float32),
        compiler_params=pltpu.CompilerParams(
            dimension_semantics=("arbitrary", "arbitrary", "arbitrary"),
        ),
    )(xbf, w1b, w3b, w2b, wd)


def kernel(hidden_states, gate_w, w1, w2, w3):
    x = hidden_states.reshape(-1, HIDDEN)
    gate_wp = jnp.zeros((LANES, HIDDEN), jnp.float32).at[:NUM_EXPERTS].set(gate_w)
    wd, xbf, _idx, _wts = _run_router(x, gate_wp)
    w1b = w1.astype(jnp.bfloat16)
    w3b = w3.astype(jnp.bfloat16)
    w2b = w2.astype(jnp.bfloat16)
    out = _run_dense(xbf, w1b, w3b, w2b, wd)
    return out.reshape(BATCH, SEQ, HIDDEN)


# speedup vs baseline: 1.6924x; 1.5307x over previous
import functools

import jax
import jax.numpy as jnp
from jax.experimental import pallas as pl
from jax.experimental.pallas import tpu as pltpu

BATCH = 2
SEQ = 8192
HIDDEN = 1024
FFN = 3584
NUM_EXPERTS = 8
TOP_K = 2

T = BATCH * SEQ
LANES = 128


_RTM = 2048


def _router_body(x_ref, gw_ref, xbf_ref, idx_ref, wts_ref):
    x = x_ref[...]
    logits = jax.lax.dot_general(
        x, gw_ref[...], (((1,), (1,)), ((), ())),
        preferred_element_type=jnp.float32)
    lane = jax.lax.broadcasted_iota(jnp.int32, logits.shape, 1)
    neg = jnp.float32(-1e30)
    l0 = jnp.where(lane < NUM_EXPERTS, logits, neg)
    m1 = jnp.max(l0, axis=1, keepdims=True)
    i1 = jnp.min(jnp.where(l0 == m1, lane, LANES), axis=1, keepdims=True)
    l1 = jnp.where(lane == i1, neg, l0)
    m2 = jnp.max(l1, axis=1, keepdims=True)
    i2 = jnp.min(jnp.where(l1 == m2, lane, LANES), axis=1, keepdims=True)
    wa = 1.0 / (1.0 + jnp.exp(m2 - m1))
    wb = 1.0 - wa
    idx_ref[...] = jnp.where(lane == 0, i1, jnp.where(lane == 1, i2, 0))
    wts_ref[...] = jnp.where(lane == 0, wa, jnp.where(lane == 1, wb, 0.0))
    xbf_ref[...] = x.astype(jnp.bfloat16)


def _run_router(x, gate_wp):
    grid = (T // _RTM,)
    return pl.pallas_call(
        _router_body,
        grid=grid,
        in_specs=[
            pl.BlockSpec((_RTM, HIDDEN), lambda m: (m, 0)),
            pl.BlockSpec((LANES, HIDDEN), lambda m: (0, 0)),
        ],
        out_specs=[
            pl.BlockSpec((_RTM, HIDDEN), lambda m: (m, 0)),
            pl.BlockSpec((_RTM, LANES), lambda m: (m, 0)),
            pl.BlockSpec((_RTM, LANES), lambda m: (m, 0)),
        ],
        out_shape=[
            jax.ShapeDtypeStruct((T, HIDDEN), jnp.bfloat16),
            jax.ShapeDtypeStruct((T, LANES), jnp.int32),
            jax.ShapeDtypeStruct((T, LANES), jnp.float32),
        ],
    )(x, gate_wp)


_GTM = 512
_GTF = 512


def _n_tiles():
    return (TOP_K * T) // _GTM + NUM_EXPERTS


def _gmm_body(gid_ref, xs_ref, w1_ref, w3_ref, w2_ref, ys_ref):
    xb = xs_ref[...]
    acc = jnp.zeros((_GTM, HIDDEN), jnp.float32)
    for fj in range(FFN // _GTF):
        w1s = w1_ref[0, fj * _GTF:(fj + 1) * _GTF, :]
        w3s = w3_ref[0, fj * _GTF:(fj + 1) * _GTF, :]
        w2s = w2_ref[0, :, fj * _GTF:(fj + 1) * _GTF]
        a = jax.lax.dot_general(xb, w1s, (((1,), (1,)), ((), ())),
                                preferred_element_type=jnp.float32)
        b = jax.lax.dot_general(xb, w3s, (((1,), (1,)), ((), ())),
                                preferred_element_type=jnp.float32)
        h = (a * jax.nn.sigmoid(a) * b).astype(jnp.bfloat16)
        acc = acc + jax.lax.dot_general(h, w2s, (((1,), (1,)), ((), ())),
                                        preferred_element_type=jnp.float32)
    ys_ref[...] = acc


def _run_gmm(gid, xs, w1b, w3b, w2b):
    nt = _n_tiles()
    grid_spec = pltpu.PrefetchScalarGridSpec(
        num_scalar_prefetch=1,
        grid=(nt,),
        in_specs=[
            pl.BlockSpec((_GTM, HIDDEN), lambda m, gid_ref: (m, 0)),
            pl.BlockSpec((1, FFN, HIDDEN), lambda m, gid_ref: (gid_ref[m], 0, 0)),
            pl.BlockSpec((1, FFN, HIDDEN), lambda m, gid_ref: (gid_ref[m], 0, 0)),
            pl.BlockSpec((1, HIDDEN, FFN), lambda m, gid_ref: (gid_ref[m], 0, 0)),
        ],
        out_specs=pl.BlockSpec((_GTM, HIDDEN), lambda m, gid_ref: (m, 0)),
    )
    return pl.pallas_call(
        _gmm_body,
        grid_spec=grid_spec,
        out_shape=jax.ShapeDtypeStruct((nt * _GTM, HIDDEN), jnp.float32),
        compiler_params=pltpu.CompilerParams(
            dimension_semantics=("arbitrary",),
        ),
    )(gid, xs, w1b, w3b, w2b)


def kernel(hidden_states, gate_w, w1, w2, w3):
    x = hidden_states.reshape(-1, HIDDEN)
    gate_wp = jnp.zeros((LANES, HIDDEN), jnp.float32).at[:NUM_EXPERTS].set(gate_w)
    xbf, idx_out, wts_out = _run_router(x, gate_wp)
    idx2 = idx_out[:, :TOP_K]
    wts2 = wts_out[:, :TOP_K]

    s = TOP_K * T
    m_pad = _n_tiles() * _GTM
    eid = idx2.reshape(s)
    order = jnp.argsort(eid, stable=True)
    sorted_eid = eid[order]
    counts = jnp.bincount(eid, length=NUM_EXPERTS)
    start = jnp.concatenate([jnp.zeros((1,), counts.dtype), jnp.cumsum(counts)[:-1]])
    pcnt = ((counts + _GTM - 1) // _GTM) * _GTM
    poff = jnp.concatenate([jnp.zeros((1,), pcnt.dtype), jnp.cumsum(pcnt)[:-1]])
    rank = jnp.arange(s) - start[sorted_eid]
    dest = (poff[sorted_eid] + rank).astype(jnp.int32)
    sorted_tok = jnp.zeros((m_pad,), jnp.int32).at[dest].set(
        (order // TOP_K).astype(jnp.int32))
    inv = jnp.zeros((s,), jnp.int32).at[order].set(dest)
    tile_start = jnp.arange(_n_tiles()) * _GTM
    poff_end = jnp.cumsum(pcnt)
    gid = jnp.clip(jnp.sum(tile_start[:, None] >= poff_end[None, :], axis=1),
                   0, NUM_EXPERTS - 1).astype(jnp.int32)

    xs = xbf[sorted_tok]
    w1b = w1.astype(jnp.bfloat16)
    w3b = w3.astype(jnp.bfloat16)
    w2b = w2.astype(jnp.bfloat16)
    ys = _run_gmm(gid, xs, w1b, w3b, w2b)

    pos_a = inv[0::TOP_K]
    pos_b = inv[1::TOP_K]
    final = wts2[:, 0:1] * ys[pos_a] + wts2[:, 1:2] * ys[pos_b]
    return final.reshape(BATCH, SEQ, HIDDEN)
